# baseline (speedup 1.0000x reference)
import jax
import jax.numpy as jnp
from jax.experimental import pallas as pl
from jax.experimental.pallas import tpu as pltpu

_HEIGHT = 4096
_WIDTH = 4096
_BATCH = 2048
_NBLK = 512
_XBLK = 1024
_MBLK = 512


def _decode_matmul_kernel(x_ref, code_ref, mm_ref, out_ref, w_ref):
    m = pl.program_id(1)

    @pl.when(m == 0)
    def _decode():
        code = code_ref[...]
        mins = mm_ref[:, 0:1]
        maxs = mm_ref[:, 1:2]
        w_ref[...] = (mins * (code == 1).astype(jnp.bfloat16)
                      + maxs * (code == 2).astype(jnp.bfloat16))

    w = w_ref[...]
    for mb in range(0, _XBLK, _MBLK):
        out_ref[mb:mb + _MBLK, :] = jax.lax.dot_general(
            x_ref[mb:mb + _MBLK, :],
            w,
            (((1,), (1,)), ((), ())),
            preferred_element_type=jnp.float32,
        ).astype(jnp.bfloat16)


def kernel(x, w_tern, ter_minmax):
    mm = ter_minmax.reshape(_HEIGHT, 2)
    nj = _HEIGHT // _NBLK
    nm = _BATCH // _XBLK
    return pl.pallas_call(
        _decode_matmul_kernel,
        grid=(nj, nm),
        in_specs=[
            pl.BlockSpec((_XBLK, _WIDTH), lambda j, m: (m, 0)),
            pl.BlockSpec((_NBLK, _WIDTH), lambda j, m: (j, 0)),
            pl.BlockSpec((_NBLK, 2), lambda j, m: (j, 0)),
        ],
        out_specs=pl.BlockSpec((_XBLK, _NBLK), lambda j, m: (m, j)),
        out_shape=jax.ShapeDtypeStruct((_BATCH, _HEIGHT), jnp.bfloat16),
        scratch_shapes=[pltpu.VMEM((_NBLK, _WIDTH), jnp.bfloat16)],
    )(x, w_tern, mm)


# speedup vs baseline: 1.3282x; 1.3282x over previous
import jax
import jax.numpy as jnp
from jax.experimental import pallas as pl

_HEIGHT = 4096
_WIDTH = 4096
_BATCH = 2048
_NBLK = 512
_NSUB = 256


def _decode_matmul_kernel(x_ref, code_ref, mm_ref, out_ref):
    code = code_ref[...]
    mins = mm_ref[:, 0:1]
    maxs = mm_ref[:, 1:2]
    w = (mins * (code == 1).astype(jnp.bfloat16)
         + maxs * (code == 2).astype(jnp.bfloat16))
    for nb in range(0, _NBLK, _NSUB):
        out_ref[:, nb:nb + _NSUB] = jax.lax.dot_general(
            x_ref[...],
            w[nb:nb + _NSUB, :],
            (((1,), (1,)), ((), ())),
            preferred_element_type=jnp.float32,
        ).astype(jnp.bfloat16)


def kernel(x, w_tern, ter_minmax):
    mm = ter_minmax.reshape(_HEIGHT, 2)
    nj = _HEIGHT // _NBLK
    return pl.pallas_call(
        _decode_matmul_kernel,
        grid=(nj,),
        in_specs=[
            pl.BlockSpec((_BATCH, _WIDTH), lambda j: (0, 0)),
            pl.BlockSpec((_NBLK, _WIDTH), lambda j: (j, 0)),
            pl.BlockSpec((_NBLK, 2), lambda j: (j, 0)),
        ],
        out_specs=pl.BlockSpec((_BATCH, _NBLK), lambda j: (0, j)),
        out_shape=jax.ShapeDtypeStruct((_BATCH, _HEIGHT), jnp.bfloat16),
    )(x, w_tern, mm)
